# baseline (speedup 1.0000x reference)
import functools

import jax
import jax.numpy as jnp
from jax import lax
from jax.experimental import pallas as pl
from jax.experimental.pallas import tpu as pltpu
from jax.experimental.pallas import tpu_sc as plsc

K_EDGES = 30
N_NODES = 256
N_BATCH = 8
N_LAG = 96
T_LEN = 2000

SC_CORES = 2
SC_SUBCORES = 16
SC_LANES = 16
N_WORKERS = SC_CORES * SC_SUBCORES
ROWS_PER_W = N_NODES // N_WORKERS
VREGS_PER_ROW = N_NODES // SC_LANES

_I32_MIN = -2147483648


def _prep_body(cau_ref, cos_ref):
    c = cau_ref[...]
    cmin = jnp.min(c, axis=0, keepdims=True)
    cmax = jnp.max(c, axis=0, keepdims=True)
    cau = (c - cmin) / (cmax - cmin + 1e-8)
    ct = jnp.transpose(cau, (1, 0))
    g = jnp.dot(ct, cau, preferred_element_type=jnp.float32)
    nsq = jnp.sum(ct * ct, axis=1, keepdims=True)
    norms_r = jnp.sqrt(nsq)
    norms_c = jnp.transpose(norms_r, (1, 0))
    cos = g / (norms_r * norms_c + 1e-8)
    row_i = lax.broadcasted_iota(jnp.int32, (N_NODES, N_NODES), 0)
    col_i = lax.broadcasted_iota(jnp.int32, (N_NODES, N_NODES), 1)
    cos_ref[...] = cos - jnp.where(row_i == col_i, 1.0, 0.0).astype(jnp.float32)


def _prep(cau_data):
    return pl.pallas_call(
        _prep_body,
        out_shape=jax.ShapeDtypeStruct((N_NODES, N_NODES), jnp.float32),
    )(cau_data)


def _fill_body(bx_ref, blz_ref, bxT_ref):
    i = pl.program_id(0)
    blz_ref[...] = jnp.zeros((N_NODES, N_BATCH * N_NODES), jnp.float32)

    @pl.when(i == 0)
    def _():
        for b in range(N_BATCH):
            bxT_ref[b] = jnp.transpose(bx_ref[b], (1, 0))


def _fill(batch_x):
    return pl.pallas_call(
        _fill_body,
        grid=(N_BATCH,),
        in_specs=[pl.BlockSpec((N_BATCH, N_LAG, N_NODES), lambda i: (0, 0, 0))],
        out_specs=(
            pl.BlockSpec((N_NODES, N_BATCH * N_NODES), lambda i: (i, 0)),
            pl.BlockSpec((N_BATCH, N_NODES, N_LAG), lambda i: (0, 0, 0)),
        ),
        out_shape=(
            jax.ShapeDtypeStruct((N_BATCH * N_NODES, N_BATCH * N_NODES),
                                 jnp.float32),
            jax.ShapeDtypeStruct((N_BATCH, N_NODES, N_LAG), jnp.float32),
        ),
    )(batch_x)


def _f32_sortable_key(v):
    s = lax.bitcast_convert_type(v, jnp.int32)
    return jnp.where(s < 0, jnp.int32(_I32_MIN) - s, s)


_KEY_BOUND = 1069547520


def _rev(x):
    return lax.rev(x, (0,))


def _sort(x):
    return lax.sort(x, dimension=0)


def _merge_sorted_pair(a, b):
    rb = _rev(b)
    return _sort(jnp.minimum(a, rb)), _sort(jnp.maximum(a, rb))


def _top32_merge(a, b):
    alo, ahi = a
    blo, bhi = b
    t0 = jnp.maximum(alo, _rev(bhi))
    t1 = jnp.maximum(ahi, _rev(blo))
    return _merge_sorted_pair(_sort(t0), _sort(t1))


def _topk_row_adj(keys):
    sorted_vregs = [_sort(k) for k in keys]
    pairs = [
        _merge_sorted_pair(sorted_vregs[2 * p], sorted_vregs[2 * p + 1])
        for p in range(VREGS_PER_ROW // 2)
    ]
    while len(pairs) > 1:
        pairs = [
            _top32_merge(pairs[2 * p], pairs[2 * p + 1])
            for p in range(len(pairs) // 2)
        ]
    top_lo, _ = pairs[0]

    pos = 2 * SC_LANES - K_EDGES
    iota = lax.iota(jnp.int32, SC_LANES)
    nb = jnp.full((SC_LANES,), -_KEY_BOUND, dtype=jnp.int32)
    thr = jnp.max(jnp.where(iota == pos, top_lo, nb))
    thr_v = jnp.full((SC_LANES,), thr, dtype=jnp.int32)

    k_splat = jnp.full((SC_LANES,), K_EDGES, dtype=jnp.int32)
    n_gt = plsc.all_reduce_population_count(keys[0] > thr_v)
    for i in range(1, VREGS_PER_ROW):
        n_gt = n_gt + plsc.all_reduce_population_count(keys[i] > thr_v)
    need = k_splat - n_gt
    out = []
    base = jnp.zeros((SC_LANES,), dtype=jnp.int32)
    for i in range(VREGS_PER_ROW):
        gt = keys[i] > thr_v
        eq = keys[i] == thr_v
        pref = plsc.cumsum(jnp.where(eq, 1, 0).astype(jnp.int32))
        take = eq & ((base + pref) <= need)
        base = base + plsc.all_reduce_population_count(eq)
        out.append(jnp.where(gt | take, 1.0, 0.0).astype(jnp.float32))
    return out


def _topk_body(cos_hbm, adj_hbm, rows_v, adj_v, sem):
    wid = lax.axis_index("s") * SC_CORES + lax.axis_index("c")
    row0 = wid * ROWS_PER_W
    pltpu.sync_copy(cos_hbm.at[pl.ds(row0, ROWS_PER_W)], rows_v)
    for r in range(ROWS_PER_W):
        keys = [
            _f32_sortable_key(rows_v[r, pl.ds(i * SC_LANES, SC_LANES)])
            for i in range(VREGS_PER_ROW)
        ]
        adj_row = _topk_row_adj(keys)
        for i in range(VREGS_PER_ROW):
            adj_v[r, pl.ds(i * SC_LANES, SC_LANES)] = adj_row[i]
    pltpu.sync_copy(adj_v, adj_hbm.at[pl.ds(row0, ROWS_PER_W)])


def _topk_adj(cos):
    mesh = plsc.VectorSubcoreMesh(
        core_axis_name="c", subcore_axis_name="s",
        num_cores=SC_CORES, num_subcores=SC_SUBCORES)
    krn = pl.kernel(
        _topk_body,
        out_type=jax.ShapeDtypeStruct((N_NODES, N_NODES), jnp.float32),
        mesh=mesh,
        compiler_params=pltpu.CompilerParams(needs_layout_passes=False),
        scratch_types=[
            pltpu.VMEM((ROWS_PER_W, N_NODES), jnp.float32),
            pltpu.VMEM((ROWS_PER_W, N_NODES), jnp.float32),
            pltpu.SemaphoreType.DMA,
        ],
    )
    return krn(cos)


def _lap_diag_body(adj_ref, blz_ref, bl_ref, sl_ref, l_ref):
    del blz_ref
    i = pl.program_id(0)

    @pl.when(i == 0)
    def _():
        a = adj_ref[...]
        a = jnp.maximum(a, jnp.transpose(a, (1, 0)))
        row_i = lax.broadcasted_iota(jnp.int32, (N_NODES, N_NODES), 0)
        col_i = lax.broadcasted_iota(jnp.int32, (N_NODES, N_NODES), 1)
        eye = jnp.where(row_i == col_i, 1.0, 0.0).astype(jnp.float32)
        sl = jnp.maximum(a, eye)
        sl_ref[...] = sl
        deg = jnp.sum(sl, axis=1, keepdims=True)
        dinv_r = jnp.where(deg > 0, lax.rsqrt(deg), 0.0)
        dinv_c = jnp.transpose(dinv_r, (1, 0))
        l_ref[...] = eye - dinv_r * sl * dinv_c

    bl_ref[...] = l_ref[...]


def _lap_diag(adj, bl_zeros):
    return pl.pallas_call(
        _lap_diag_body,
        grid=(N_BATCH,),
        in_specs=[
            pl.BlockSpec((N_NODES, N_NODES), lambda i: (0, 0)),
            pl.BlockSpec((N_NODES, N_NODES), lambda i: (0, 0)),
        ],
        out_specs=(
            pl.BlockSpec((N_NODES, N_NODES), lambda i: (i, i)),
            pl.BlockSpec((N_NODES, N_NODES), lambda i: (0, 0)),
            pl.BlockSpec((N_NODES, N_NODES), lambda i: (0, 0)),
        ),
        out_shape=(
            jax.ShapeDtypeStruct((N_BATCH * N_NODES, N_BATCH * N_NODES),
                                 jnp.float32),
            jax.ShapeDtypeStruct((N_NODES, N_NODES), jnp.float32),
            jax.ShapeDtypeStruct((N_NODES, N_NODES), jnp.float32),
        ),
        input_output_aliases={1: 0},
    )(adj, bl_zeros)


@jax.jit
def kernel(batch_x, last_edge_index, cau_data):
    del last_edge_index
    cos = _prep(cau_data)
    adj = _topk_adj(cos)
    bl_zeros, bxT = _fill(batch_x)
    batch_l, selfloop_adj, l_sym = _lap_diag(adj, bl_zeros)
    pyg_x = bxT.reshape(N_BATCH * N_NODES, N_LAG)
    return (pyg_x, selfloop_adj, batch_l, l_sym, bxT)


# speedup vs baseline: 1.0271x; 1.0271x over previous
import functools

import jax
import jax.numpy as jnp
from jax import lax
from jax.experimental import pallas as pl
from jax.experimental.pallas import tpu as pltpu
from jax.experimental.pallas import tpu_sc as plsc

K_EDGES = 30
N_NODES = 256
N_BATCH = 8
N_LAG = 96
T_LEN = 2000

SC_CORES = 2
SC_SUBCORES = 16
SC_LANES = 16
N_WORKERS = SC_CORES * SC_SUBCORES
ROWS_PER_W = N_NODES // N_WORKERS
VREGS_PER_ROW = N_NODES // SC_LANES

_I32_MIN = -2147483648


def _prep_body(cau_ref, bx_ref, cos_ref, blz_ref, bxT_ref):
    i = pl.program_id(0)
    blz_ref[...] = jnp.zeros((N_NODES, N_BATCH * N_NODES), jnp.float32)

    @pl.when(i == 0)
    def _():
        c = cau_ref[...]
        cmin = jnp.min(c, axis=0, keepdims=True)
        cmax = jnp.max(c, axis=0, keepdims=True)
        cau = (c - cmin) / (cmax - cmin + 1e-8)
        ct = jnp.transpose(cau, (1, 0))
        g = jnp.dot(ct, cau, preferred_element_type=jnp.float32)
        nsq = jnp.sum(ct * ct, axis=1, keepdims=True)
        norms_r = jnp.sqrt(nsq)
        norms_c = jnp.transpose(norms_r, (1, 0))
        cos = g / (norms_r * norms_c + 1e-8)
        row_i = lax.broadcasted_iota(jnp.int32, (N_NODES, N_NODES), 0)
        col_i = lax.broadcasted_iota(jnp.int32, (N_NODES, N_NODES), 1)
        cos_ref[...] = cos - jnp.where(row_i == col_i, 1.0, 0.0).astype(
            jnp.float32)
        for b in range(N_BATCH):
            bxT_ref[b] = jnp.transpose(bx_ref[b], (1, 0))


def _prep(cau_data, batch_x):
    return pl.pallas_call(
        _prep_body,
        grid=(N_BATCH,),
        in_specs=[
            pl.BlockSpec((T_LEN, N_NODES), lambda i: (0, 0)),
            pl.BlockSpec((N_BATCH, N_LAG, N_NODES), lambda i: (0, 0, 0)),
        ],
        out_specs=(
            pl.BlockSpec((N_NODES, N_NODES), lambda i: (0, 0)),
            pl.BlockSpec((N_NODES, N_BATCH * N_NODES), lambda i: (i, 0)),
            pl.BlockSpec((N_BATCH, N_NODES, N_LAG), lambda i: (0, 0, 0)),
        ),
        out_shape=(
            jax.ShapeDtypeStruct((N_NODES, N_NODES), jnp.float32),
            jax.ShapeDtypeStruct((N_BATCH * N_NODES, N_BATCH * N_NODES),
                                 jnp.float32),
            jax.ShapeDtypeStruct((N_BATCH, N_NODES, N_LAG), jnp.float32),
        ),
    )(cau_data, batch_x)


def _f32_sortable_key(v):
    s = lax.bitcast_convert_type(v, jnp.int32)
    return jnp.where(s < 0, jnp.int32(_I32_MIN) - s, s)


_KEY_BOUND = 1069547520


def _rev(x):
    return lax.rev(x, (0,))


def _sort(x):
    return lax.sort(x, dimension=0)


def _merge_sorted_pair(a, b):
    rb = _rev(b)
    return _sort(jnp.minimum(a, rb)), _sort(jnp.maximum(a, rb))


def _top32_merge(a, b):
    alo, ahi = a
    blo, bhi = b
    t0 = jnp.maximum(alo, _rev(bhi))
    t1 = jnp.maximum(ahi, _rev(blo))
    return _merge_sorted_pair(_sort(t0), _sort(t1))


def _topk_row_adj(keys):
    sorted_vregs = [_sort(k) for k in keys]
    pairs = [
        _merge_sorted_pair(sorted_vregs[2 * p], sorted_vregs[2 * p + 1])
        for p in range(VREGS_PER_ROW // 2)
    ]
    while len(pairs) > 1:
        pairs = [
            _top32_merge(pairs[2 * p], pairs[2 * p + 1])
            for p in range(len(pairs) // 2)
        ]
    top_lo, _ = pairs[0]

    pos = 2 * SC_LANES - K_EDGES
    iota = lax.iota(jnp.int32, SC_LANES)
    nb = jnp.full((SC_LANES,), -_KEY_BOUND, dtype=jnp.int32)
    thr = jnp.max(jnp.where(iota == pos, top_lo, nb))
    thr_v = jnp.full((SC_LANES,), thr, dtype=jnp.int32)

    k_splat = jnp.full((SC_LANES,), K_EDGES, dtype=jnp.int32)
    n_gt = plsc.all_reduce_population_count(keys[0] > thr_v)
    for i in range(1, VREGS_PER_ROW):
        n_gt = n_gt + plsc.all_reduce_population_count(keys[i] > thr_v)
    need = k_splat - n_gt
    out = []
    base = jnp.zeros((SC_LANES,), dtype=jnp.int32)
    for i in range(VREGS_PER_ROW):
        gt = keys[i] > thr_v
        eq = keys[i] == thr_v
        pref = plsc.cumsum(jnp.where(eq, 1, 0).astype(jnp.int32))
        take = eq & ((base + pref) <= need)
        base = base + plsc.all_reduce_population_count(eq)
        out.append(jnp.where(gt | take, 1.0, 0.0).astype(jnp.float32))
    return out


def _topk_body(cos_hbm, adj_hbm, rows_v, adj_v, sem):
    wid = lax.axis_index("s") * SC_CORES + lax.axis_index("c")
    row0 = wid * ROWS_PER_W
    pltpu.sync_copy(cos_hbm.at[pl.ds(row0, ROWS_PER_W)], rows_v)
    for r in range(ROWS_PER_W):
        keys = [
            _f32_sortable_key(rows_v[r, pl.ds(i * SC_LANES, SC_LANES)])
            for i in range(VREGS_PER_ROW)
        ]
        adj_row = _topk_row_adj(keys)
        for i in range(VREGS_PER_ROW):
            adj_v[r, pl.ds(i * SC_LANES, SC_LANES)] = adj_row[i]
    pltpu.sync_copy(adj_v, adj_hbm.at[pl.ds(row0, ROWS_PER_W)])


def _topk_adj(cos):
    mesh = plsc.VectorSubcoreMesh(
        core_axis_name="c", subcore_axis_name="s",
        num_cores=SC_CORES, num_subcores=SC_SUBCORES)
    krn = pl.kernel(
        _topk_body,
        out_type=jax.ShapeDtypeStruct((N_NODES, N_NODES), jnp.float32),
        mesh=mesh,
        compiler_params=pltpu.CompilerParams(needs_layout_passes=False),
        scratch_types=[
            pltpu.VMEM((ROWS_PER_W, N_NODES), jnp.float32),
            pltpu.VMEM((ROWS_PER_W, N_NODES), jnp.float32),
            pltpu.SemaphoreType.DMA,
        ],
    )
    return krn(cos)


def _lap_diag_body(adj_ref, blz_ref, bl_ref, sl_ref, l_ref):
    del blz_ref
    i = pl.program_id(0)

    @pl.when(i == 0)
    def _():
        a = adj_ref[...]
        a = jnp.maximum(a, jnp.transpose(a, (1, 0)))
        row_i = lax.broadcasted_iota(jnp.int32, (N_NODES, N_NODES), 0)
        col_i = lax.broadcasted_iota(jnp.int32, (N_NODES, N_NODES), 1)
        eye = jnp.where(row_i == col_i, 1.0, 0.0).astype(jnp.float32)
        sl = jnp.maximum(a, eye)
        sl_ref[...] = sl
        deg = jnp.sum(sl, axis=1, keepdims=True)
        dinv_r = jnp.where(deg > 0, lax.rsqrt(deg), 0.0)
        dinv_c = jnp.transpose(dinv_r, (1, 0))
        l_ref[...] = eye - dinv_r * sl * dinv_c

    bl_ref[...] = l_ref[...]


def _lap_diag(adj, bl_zeros):
    return pl.pallas_call(
        _lap_diag_body,
        grid=(N_BATCH,),
        in_specs=[
            pl.BlockSpec((N_NODES, N_NODES), lambda i: (0, 0)),
            pl.BlockSpec((N_NODES, N_NODES), lambda i: (0, 0)),
        ],
        out_specs=(
            pl.BlockSpec((N_NODES, N_NODES), lambda i: (i, i)),
            pl.BlockSpec((N_NODES, N_NODES), lambda i: (0, 0)),
            pl.BlockSpec((N_NODES, N_NODES), lambda i: (0, 0)),
        ),
        out_shape=(
            jax.ShapeDtypeStruct((N_BATCH * N_NODES, N_BATCH * N_NODES),
                                 jnp.float32),
            jax.ShapeDtypeStruct((N_NODES, N_NODES), jnp.float32),
            jax.ShapeDtypeStruct((N_NODES, N_NODES), jnp.float32),
        ),
        input_output_aliases={1: 0},
    )(adj, bl_zeros)


@jax.jit
def kernel(batch_x, last_edge_index, cau_data):
    del last_edge_index
    cos, bl_zeros, bxT = _prep(cau_data, batch_x)
    adj = _topk_adj(cos)
    batch_l, selfloop_adj, l_sym = _lap_diag(adj, bl_zeros)
    pyg_x = bxT.reshape(N_BATCH * N_NODES, N_LAG)
    return (pyg_x, selfloop_adj, batch_l, l_sym, bxT)


# speedup vs baseline: 1.0774x; 1.0490x over previous
import functools

import jax
import jax.numpy as jnp
from jax import lax
from jax.experimental import pallas as pl
from jax.experimental.pallas import tpu as pltpu
from jax.experimental.pallas import tpu_sc as plsc

K_EDGES = 30
N_NODES = 256
N_BATCH = 8
N_LAG = 96
T_LEN = 2000

SC_CORES = 2
SC_SUBCORES = 16
SC_LANES = 16
N_WORKERS = SC_CORES * SC_SUBCORES
ROWS_PER_W = N_NODES // N_WORKERS
VREGS_PER_ROW = N_NODES // SC_LANES

_I32_MIN = -2147483648


def _prep_body(cau_ref, bx_ref, cos_ref, bxT_ref):
    c = cau_ref[...]
    cmin = jnp.min(c, axis=0, keepdims=True)
    cmax = jnp.max(c, axis=0, keepdims=True)
    cau = (c - cmin) / (cmax - cmin + 1e-8)
    ct = jnp.transpose(cau, (1, 0))
    g = jnp.dot(ct, cau, preferred_element_type=jnp.float32)
    nsq = jnp.sum(ct * ct, axis=1, keepdims=True)
    norms_r = jnp.sqrt(nsq)
    norms_c = jnp.transpose(norms_r, (1, 0))
    cos = g / (norms_r * norms_c + 1e-8)
    row_i = lax.broadcasted_iota(jnp.int32, (N_NODES, N_NODES), 0)
    col_i = lax.broadcasted_iota(jnp.int32, (N_NODES, N_NODES), 1)
    cos_ref[...] = cos - jnp.where(row_i == col_i, 1.0, 0.0).astype(
        jnp.float32)
    for b in range(N_BATCH):
        bxT_ref[b] = jnp.transpose(bx_ref[b], (1, 0))


def _prep(cau_data, batch_x):
    return pl.pallas_call(
        _prep_body,
        out_shape=(
            jax.ShapeDtypeStruct((N_NODES, N_NODES), jnp.float32),
            jax.ShapeDtypeStruct((N_BATCH, N_NODES, N_LAG), jnp.float32),
        ),
    )(cau_data, batch_x)


def _f32_sortable_key(v):
    s = lax.bitcast_convert_type(v, jnp.int32)
    return jnp.where(s < 0, jnp.int32(_I32_MIN) - s, s)


_KEY_BOUND = 1069547520


def _rev(x):
    return lax.rev(x, (0,))


def _sort(x):
    return lax.sort(x, dimension=0)


def _merge_sorted_pair(a, b):
    rb = _rev(b)
    return _sort(jnp.minimum(a, rb)), _sort(jnp.maximum(a, rb))


def _top32_merge(a, b):
    alo, ahi = a
    blo, bhi = b
    t0 = jnp.maximum(alo, _rev(bhi))
    t1 = jnp.maximum(ahi, _rev(blo))
    return _merge_sorted_pair(_sort(t0), _sort(t1))


def _topk_row_adj(keys):
    sorted_vregs = [_sort(k) for k in keys]
    pairs = [
        _merge_sorted_pair(sorted_vregs[2 * p], sorted_vregs[2 * p + 1])
        for p in range(VREGS_PER_ROW // 2)
    ]
    while len(pairs) > 1:
        pairs = [
            _top32_merge(pairs[2 * p], pairs[2 * p + 1])
            for p in range(len(pairs) // 2)
        ]
    top_lo, _ = pairs[0]

    pos = 2 * SC_LANES - K_EDGES
    iota = lax.iota(jnp.int32, SC_LANES)
    nb = jnp.full((SC_LANES,), -_KEY_BOUND, dtype=jnp.int32)
    thr = jnp.max(jnp.where(iota == pos, top_lo, nb))
    thr_v = jnp.full((SC_LANES,), thr, dtype=jnp.int32)

    k_splat = jnp.full((SC_LANES,), K_EDGES, dtype=jnp.int32)
    n_gt = plsc.all_reduce_population_count(keys[0] > thr_v)
    for i in range(1, VREGS_PER_ROW):
        n_gt = n_gt + plsc.all_reduce_population_count(keys[i] > thr_v)
    need = k_splat - n_gt
    out = []
    base = jnp.zeros((SC_LANES,), dtype=jnp.int32)
    for i in range(VREGS_PER_ROW):
        gt = keys[i] > thr_v
        eq = keys[i] == thr_v
        pref = plsc.cumsum(jnp.where(eq, 1, 0).astype(jnp.int32))
        take = eq & ((base + pref) <= need)
        base = base + plsc.all_reduce_population_count(eq)
        out.append(jnp.where(gt | take, 1.0, 0.0).astype(jnp.float32))
    return out


def _topk_body(cos_hbm, adj_hbm, rows_v, adj_v, sem):
    wid = lax.axis_index("s") * SC_CORES + lax.axis_index("c")
    row0 = wid * ROWS_PER_W
    pltpu.sync_copy(cos_hbm.at[pl.ds(row0, ROWS_PER_W)], rows_v)

    def row_step(r, carry):
        keys = [
            _f32_sortable_key(rows_v[r, pl.ds(i * SC_LANES, SC_LANES)])
            for i in range(VREGS_PER_ROW)
        ]
        adj_row = _topk_row_adj(keys)
        for i in range(VREGS_PER_ROW):
            adj_v[r, pl.ds(i * SC_LANES, SC_LANES)] = adj_row[i]
        return carry

    lax.fori_loop(0, ROWS_PER_W, row_step, 0)
    pltpu.sync_copy(adj_v, adj_hbm.at[pl.ds(row0, ROWS_PER_W)])


def _topk_adj(cos):
    mesh = plsc.VectorSubcoreMesh(
        core_axis_name="c", subcore_axis_name="s",
        num_cores=SC_CORES, num_subcores=SC_SUBCORES)
    krn = pl.kernel(
        _topk_body,
        out_type=jax.ShapeDtypeStruct((N_NODES, N_NODES), jnp.float32),
        mesh=mesh,
        compiler_params=pltpu.CompilerParams(needs_layout_passes=False),
        scratch_types=[
            pltpu.VMEM((ROWS_PER_W, N_NODES), jnp.float32),
            pltpu.VMEM((ROWS_PER_W, N_NODES), jnp.float32),
            pltpu.SemaphoreType.DMA,
        ],
    )
    return krn(cos)


def _lap_kron_body(adj_ref, bl_ref, sl_ref, l_ref):
    i = pl.program_id(0)

    @pl.when(i == 0)
    def _():
        a = adj_ref[...]
        a = jnp.maximum(a, jnp.transpose(a, (1, 0)))
        row_i = lax.broadcasted_iota(jnp.int32, (N_NODES, N_NODES), 0)
        col_i = lax.broadcasted_iota(jnp.int32, (N_NODES, N_NODES), 1)
        eye = jnp.where(row_i == col_i, 1.0, 0.0).astype(jnp.float32)
        sl = jnp.maximum(a, eye)
        sl_ref[...] = sl
        deg = jnp.sum(sl, axis=1, keepdims=True)
        dinv_r = jnp.where(deg > 0, lax.rsqrt(deg), 0.0)
        dinv_c = jnp.transpose(dinv_r, (1, 0))
        l_ref[...] = eye - dinv_r * sl * dinv_c

    for j in range(N_BATCH):
        blk = jnp.where(j == i, l_ref[...], jnp.zeros((N_NODES, N_NODES),
                                                      jnp.float32))
        bl_ref[:, pl.ds(j * N_NODES, N_NODES)] = blk


def _lap_kron(adj):
    return pl.pallas_call(
        _lap_kron_body,
        grid=(N_BATCH,),
        in_specs=[pl.BlockSpec((N_NODES, N_NODES), lambda i: (0, 0))],
        out_specs=(
            pl.BlockSpec((N_NODES, N_BATCH * N_NODES), lambda i: (i, 0)),
            pl.BlockSpec((N_NODES, N_NODES), lambda i: (0, 0)),
            pl.BlockSpec((N_NODES, N_NODES), lambda i: (0, 0)),
        ),
        out_shape=(
            jax.ShapeDtypeStruct((N_BATCH * N_NODES, N_BATCH * N_NODES),
                                 jnp.float32),
            jax.ShapeDtypeStruct((N_NODES, N_NODES), jnp.float32),
            jax.ShapeDtypeStruct((N_NODES, N_NODES), jnp.float32),
        ),
    )(adj)


@jax.jit
def kernel(batch_x, last_edge_index, cau_data):
    del last_edge_index
    cos, bxT = _prep(cau_data, batch_x)
    adj = _topk_adj(cos)
    batch_l, selfloop_adj, l_sym = _lap_kron(adj)
    pyg_x = bxT.reshape(N_BATCH * N_NODES, N_LAG)
    return (pyg_x, selfloop_adj, batch_l, l_sym, bxT)


# speedup vs baseline: 1.0782x; 1.0007x over previous
import functools

import jax
import jax.numpy as jnp
from jax import lax
from jax.experimental import pallas as pl
from jax.experimental.pallas import tpu as pltpu
from jax.experimental.pallas import tpu_sc as plsc

K_EDGES = 30
N_NODES = 256
N_BATCH = 8
N_LAG = 96
T_LEN = 2000

SC_CORES = 2
SC_SUBCORES = 16
SC_LANES = 16
N_WORKERS = SC_CORES * SC_SUBCORES
ROWS_PER_W = N_NODES // N_WORKERS
VREGS_PER_ROW = N_NODES // SC_LANES

_I32_MIN = -2147483648


def _prep_body(cau_ref, bx_ref, cos_ref, bxT_ref):
    c = cau_ref[...]
    cmin = jnp.min(c, axis=0, keepdims=True)
    cmax = jnp.max(c, axis=0, keepdims=True)
    cau = (c - cmin) / (cmax - cmin + 1e-8)
    ct = jnp.transpose(cau, (1, 0))
    g = jnp.dot(ct, cau, preferred_element_type=jnp.float32)
    nsq = jnp.sum(ct * ct, axis=1, keepdims=True)
    norms_r = jnp.sqrt(nsq)
    norms_c = jnp.transpose(norms_r, (1, 0))
    cos = g / (norms_r * norms_c + 1e-8)
    row_i = lax.broadcasted_iota(jnp.int32, (N_NODES, N_NODES), 0)
    col_i = lax.broadcasted_iota(jnp.int32, (N_NODES, N_NODES), 1)
    cos_ref[...] = cos - jnp.where(row_i == col_i, 1.0, 0.0).astype(
        jnp.float32)
    for b in range(N_BATCH):
        bxT_ref[b] = jnp.transpose(bx_ref[b], (1, 0))


def _prep(cau_data, batch_x):
    return pl.pallas_call(
        _prep_body,
        out_shape=(
            jax.ShapeDtypeStruct((N_NODES, N_NODES), jnp.float32),
            jax.ShapeDtypeStruct((N_BATCH, N_NODES, N_LAG), jnp.float32),
        ),
    )(cau_data, batch_x)


def _f32_sortable_key(v):
    s = lax.bitcast_convert_type(v, jnp.int32)
    return jnp.where(s < 0, jnp.int32(_I32_MIN) - s, s)


_KEY_BOUND = 1069547520


def _rev(x):
    return lax.rev(x, (0,))


def _sort(x):
    return lax.sort(x, dimension=0)


def _merge_sorted_pair(a, b):
    rb = _rev(b)
    return _sort(jnp.minimum(a, rb)), _sort(jnp.maximum(a, rb))


def _top32_merge(a, b):
    alo, ahi = a
    blo, bhi = b
    t0 = jnp.maximum(alo, _rev(bhi))
    t1 = jnp.maximum(ahi, _rev(blo))
    return _merge_sorted_pair(_sort(t0), _sort(t1))


def _topk_row_adj(keys):
    sorted_vregs = [_sort(k) for k in keys]
    pairs = [
        _merge_sorted_pair(sorted_vregs[2 * p], sorted_vregs[2 * p + 1])
        for p in range(VREGS_PER_ROW // 2)
    ]
    while len(pairs) > 1:
        pairs = [
            _top32_merge(pairs[2 * p], pairs[2 * p + 1])
            for p in range(len(pairs) // 2)
        ]
    top_lo, _ = pairs[0]

    pos = 2 * SC_LANES - K_EDGES
    iota = lax.iota(jnp.int32, SC_LANES)
    nb = jnp.full((SC_LANES,), -_KEY_BOUND, dtype=jnp.int32)
    thr = jnp.max(jnp.where(iota == pos, top_lo, nb))
    thr_v = jnp.full((SC_LANES,), thr, dtype=jnp.int32)

    k_splat = jnp.full((SC_LANES,), K_EDGES, dtype=jnp.int32)
    n_gt = plsc.all_reduce_population_count(keys[0] > thr_v)
    for i in range(1, VREGS_PER_ROW):
        n_gt = n_gt + plsc.all_reduce_population_count(keys[i] > thr_v)
    need = k_splat - n_gt
    out = []
    base = jnp.zeros((SC_LANES,), dtype=jnp.int32)
    for i in range(VREGS_PER_ROW):
        gt = keys[i] > thr_v
        eq = keys[i] == thr_v
        pref = plsc.cumsum(jnp.where(eq, 1, 0).astype(jnp.int32))
        take = eq & ((base + pref) <= need)
        base = base + plsc.all_reduce_population_count(eq)
        out.append(jnp.where(gt | take, 1.0, 0.0).astype(jnp.float32))
    return out


def _topk_body(cos_hbm, adj_hbm, rows_v, adj_v, sem):
    wid = lax.axis_index("s") * SC_CORES + lax.axis_index("c")
    row0 = wid * ROWS_PER_W
    pltpu.sync_copy(cos_hbm.at[pl.ds(row0, ROWS_PER_W)], rows_v)

    def row_step(r, carry):
        keys = [
            _f32_sortable_key(rows_v[r, pl.ds(i * SC_LANES, SC_LANES)])
            for i in range(VREGS_PER_ROW)
        ]
        adj_row = _topk_row_adj(keys)
        for i in range(VREGS_PER_ROW):
            adj_v[r, pl.ds(i * SC_LANES, SC_LANES)] = adj_row[i]
        return carry

    lax.fori_loop(0, ROWS_PER_W, row_step, 0)
    pltpu.sync_copy(adj_v, adj_hbm.at[pl.ds(row0, ROWS_PER_W)])


def _topk_adj(cos):
    mesh = plsc.VectorSubcoreMesh(
        core_axis_name="c", subcore_axis_name="s",
        num_cores=SC_CORES, num_subcores=SC_SUBCORES)
    krn = pl.kernel(
        _topk_body,
        out_type=jax.ShapeDtypeStruct((N_NODES, N_NODES), jnp.float32),
        mesh=mesh,
        compiler_params=pltpu.CompilerParams(needs_layout_passes=False,
                                             use_tc_tiling_on_sc=True),
        scratch_types=[
            pltpu.VMEM((ROWS_PER_W, N_NODES), jnp.float32),
            pltpu.VMEM((ROWS_PER_W, N_NODES), jnp.float32),
            pltpu.SemaphoreType.DMA,
        ],
    )
    return krn(cos)


def _lap_kron_body(adj_ref, bl_ref, sl_ref, l_ref):
    i = pl.program_id(0)

    @pl.when(i == 0)
    def _():
        a = adj_ref[...]
        a = jnp.maximum(a, jnp.transpose(a, (1, 0)))
        row_i = lax.broadcasted_iota(jnp.int32, (N_NODES, N_NODES), 0)
        col_i = lax.broadcasted_iota(jnp.int32, (N_NODES, N_NODES), 1)
        eye = jnp.where(row_i == col_i, 1.0, 0.0).astype(jnp.float32)
        sl = jnp.maximum(a, eye)
        sl_ref[...] = sl
        deg = jnp.sum(sl, axis=1, keepdims=True)
        dinv_r = jnp.where(deg > 0, lax.rsqrt(deg), 0.0)
        dinv_c = jnp.transpose(dinv_r, (1, 0))
        l_ref[...] = eye - dinv_r * sl * dinv_c

    for j in range(N_BATCH):
        blk = jnp.where(j == i, l_ref[...], jnp.zeros((N_NODES, N_NODES),
                                                      jnp.float32))
        bl_ref[:, pl.ds(j * N_NODES, N_NODES)] = blk


def _lap_kron(adj):
    return pl.pallas_call(
        _lap_kron_body,
        grid=(N_BATCH,),
        in_specs=[pl.BlockSpec((N_NODES, N_NODES), lambda i: (0, 0))],
        out_specs=(
            pl.BlockSpec((N_NODES, N_BATCH * N_NODES), lambda i: (i, 0)),
            pl.BlockSpec((N_NODES, N_NODES), lambda i: (0, 0)),
            pl.BlockSpec((N_NODES, N_NODES), lambda i: (0, 0)),
        ),
        out_shape=(
            jax.ShapeDtypeStruct((N_BATCH * N_NODES, N_BATCH * N_NODES),
                                 jnp.float32),
            jax.ShapeDtypeStruct((N_NODES, N_NODES), jnp.float32),
            jax.ShapeDtypeStruct((N_NODES, N_NODES), jnp.float32),
        ),
    )(adj)


@jax.jit
def kernel(batch_x, last_edge_index, cau_data):
    del last_edge_index
    cos, bxT = _prep(cau_data, batch_x)
    adj = _topk_adj(cos)
    batch_l, selfloop_adj, l_sym = _lap_kron(adj)
    pyg_x = bxT.reshape(N_BATCH * N_NODES, N_LAG)
    return (pyg_x, selfloop_adj, batch_l, l_sym, bxT)
